# baseline (speedup 1.0000x reference)
import functools

import jax
import jax.numpy as jnp
from jax import lax
from jax.experimental import pallas as pl
from jax.experimental.pallas import tpu as pltpu
from jax.experimental.pallas import tpu_sc as plsc

_L = 12
_P = 100
_LEN = 5
_D = 768
_K = 5
_B = 32
_S = 2048

_G = 32
_OFFW = _L * _G
_ROWS = _K * _LEN

_NEG = -3.0e38


def _tc_body(x_ref, sk_ref, tk_ref,
             s_sim_ref, t_sim_ref, s_idx_ref, t_idx_ref,
             s_off_ref, s_red_ref, t_red_ref):
    b = pl.program_id(0)
    xb = x_ref[0]
    mean = jnp.sum(xb, axis=0, keepdims=True) * (1.0 / _S)
    n2 = jnp.sum(mean * mean, axis=1, keepdims=True)
    xn = mean * lax.rsqrt(jnp.maximum(n2, 1e-12))

    @pl.when(b == 0)
    def _():
        s_red_ref[...] = jnp.zeros((1, 1), jnp.float32)
        t_red_ref[...] = jnp.zeros((1, 1), jnp.float32)

    ii = lax.broadcasted_iota(jnp.int32, (1, _P), 1)
    ii5 = lax.broadcasted_iota(jnp.int32, (1, _K), 1)
    pos = lax.broadcasted_iota(jnp.int32, (1, _OFFW), 1)
    lfield = pos // _G
    rem = pos - lfield * _G
    kfield = rem // _LEN
    jfield = rem - kfield * _LEN

    def route(k_ref, sim_ref, idx_ref, off_ref, red_ref):
        kk = k_ref[...]
        kn2 = jnp.sum(kk * kk, axis=1, keepdims=True)
        kn = kk * lax.rsqrt(jnp.maximum(kn2, 1e-12))
        sim = lax.dot_general(xn, kn, (((1,), (1,)), ((), ())),
                              preferred_element_type=jnp.float32)
        sim_ref[pl.ds(b, 1), :] = sim
        row = sim
        racc = jnp.zeros((1, 1), jnp.float32)
        ivec = jnp.zeros((1, _K), jnp.int32)
        isel = jnp.zeros((1, _OFFW), jnp.int32)
        for k in range(_K):
            mx = jnp.max(row, axis=1, keepdims=True)
            am = jnp.min(jnp.where(row == mx, ii, _P),
                         axis=1, keepdims=True)
            ivec = jnp.where(ii5 == k, am, ivec)
            if off_ref is not None:
                isel = jnp.where(kfield == k, am, isel)
            racc = racc + mx
            row = jnp.where(ii == am, _NEG, row)
        idx_ref[pl.ds(b, 1), :] = ivec
        if off_ref is not None:
            off_ref[pl.ds(b, 1), :] = lfield * (_P * _LEN) + isel * _LEN + jfield
        red_ref[...] = red_ref[...] + racc

        @pl.when(b == _B - 1)
        def _():
            red_ref[...] = red_ref[...] * (1.0 / _B)

    route(sk_ref, s_sim_ref, s_idx_ref, s_off_ref, s_red_ref)
    route(tk_ref, t_sim_ref, t_idx_ref, None, t_red_ref)


_route_call = pl.pallas_call(
    _tc_body,
    grid=(_B,),
    in_specs=[
        pl.BlockSpec((1, _S, _D), lambda b: (b, 0, 0)),
        pl.BlockSpec((_P, _D), lambda b: (0, 0)),
        pl.BlockSpec((_P, _D), lambda b: (0, 0)),
    ],
    out_specs=[
        pl.BlockSpec((_B, _P), lambda b: (0, 0)),
        pl.BlockSpec((_B, _P), lambda b: (0, 0)),
        pl.BlockSpec((_B, _K), lambda b: (0, 0)),
        pl.BlockSpec((_B, _K), lambda b: (0, 0)),
        pl.BlockSpec((_B, _OFFW), lambda b: (0, 0)),
        pl.BlockSpec((1, 1), lambda b: (0, 0)),
        pl.BlockSpec((1, 1), lambda b: (0, 0)),
    ],
    out_shape=[
        jax.ShapeDtypeStruct((_B, _P), jnp.float32),
        jax.ShapeDtypeStruct((_B, _P), jnp.float32),
        jax.ShapeDtypeStruct((_B, _K), jnp.int32),
        jax.ShapeDtypeStruct((_B, _K), jnp.int32),
        jax.ShapeDtypeStruct((_B, _OFFW), jnp.int32),
        jax.ShapeDtypeStruct((1, 1), jnp.float32),
        jax.ShapeDtypeStruct((1, 1), jnp.float32),
    ],
)

_NC = 2
_NS = 16
_NW = _NC * _NS


@functools.cache
def _make_sc_gather():
    mesh = plsc.VectorSubcoreMesh(core_axis_name="c", subcore_axis_name="s",
                                  num_cores=_NC, num_subcores=_NS)

    @functools.partial(
        pl.kernel, mesh=mesh,
        out_type=jax.ShapeDtypeStruct((_L * _B, _G, _D), jnp.float32),
        scratch_types=[
            pltpu.VMEM((_OFFW,), jnp.int32),
            pltpu.VMEM((_G, _D), jnp.float32),
            pltpu.VMEM((_G, _D), jnp.float32),
            pltpu.SemaphoreType.DMA,
            pltpu.SemaphoreType.DMA,
        ],
    )
    def _sc_gather(tab, off_hbm, out, off_v, buf_a, buf_b, sem_a, sem_b):
        b = lax.axis_index("s") * _NC + lax.axis_index("c")

        bufs = (buf_a, buf_b)
        sems = (sem_a, sem_b)

        pltpu.sync_copy(off_hbm.at[b], off_v)
        hs = [None] * _L
        hs[0] = pltpu.async_copy(tab.at[off_v.at[pl.ds(0, _G)]],
                                 bufs[0], sems[0])
        for lyr in range(_L):
            nxt = lyr + 1
            if nxt < _L:
                hs[nxt] = pltpu.async_copy(
                    tab.at[off_v.at[pl.ds(nxt * _G, _G)]],
                    bufs[nxt % 2], sems[nxt % 2])
            hs[lyr].wait()
            pltpu.sync_copy(bufs[lyr % 2], out.at[lyr * _B + b])

    return _sc_gather


def _tgather_body(idx_ref, p0_ref, p1_ref, p2_ref, p3_ref, p4_ref, o_ref):
    del idx_ref
    for k, p_ref in enumerate((p0_ref, p1_ref, p2_ref, p3_ref, p4_ref)):
        o_ref[0, 0, pl.ds(_LEN * k, _LEN), :] = p_ref[0, 0]


def _tg_in_spec(k):
    return pl.BlockSpec((1, 1, _LEN, _D),
                        lambda l, b, idx, k=k: (l, idx[b, k], 0, 0))


_tgather_call = pl.pallas_call(
    _tgather_body,
    grid_spec=pltpu.PrefetchScalarGridSpec(
        num_scalar_prefetch=1,
        grid=(_L, _B),
        in_specs=[_tg_in_spec(k) for k in range(_K)],
        out_specs=pl.BlockSpec((1, 1, _ROWS, _D), lambda l, b, idx: (l, b, 0, 0)),
    ),
    out_shape=jax.ShapeDtypeStruct((_L, _B, _ROWS, _D), jnp.float32),
)


def kernel(x_embed, s_prompt, t_prompt, s_prompt_key, t_prompt_key):
    (s_sim, t_sim, s_idx, t_idx, s_off, s_red, t_red) = _route_call(
        x_embed, s_prompt_key, t_prompt_key)
    s_sc = _make_sc_gather()(s_prompt.reshape(_L * _P * _LEN, _D), s_off)
    t_batched = _tgather_call(t_idx, t_prompt, t_prompt, t_prompt,
                              t_prompt, t_prompt)
    s_batched = s_sc.reshape(_L, _B, _G, _D)[:, :, :_ROWS, :]
    return (s_sim, t_sim, s_idx, t_idx, s_batched, t_batched,
            s_red.reshape(()), t_red.reshape(()))
